# baseline (speedup 1.0000x reference)
import functools

import jax
import jax.numpy as jnp
from jax import lax
from jax.experimental import pallas as pl
from jax.experimental.pallas import tpu as pltpu
from jax.experimental.pallas import tpu_sc as plsc

DIM = 64
SEQ = 200
BATCH = 1024
NW = 32
SEQ_PER_W = BATCH // NW
SPLIT = 128
LANES = 16


def _make_kernel():
    mesh = plsc.VectorSubcoreMesh(core_axis_name="c", subcore_axis_name="s")

    @functools.partial(
        pl.kernel,
        mesh=mesh,
        compiler_params=pltpu.CompilerParams(use_tc_tiling_on_sc=False),
        out_type=jax.ShapeDtypeStruct((BATCH * SEQ, DIM), jnp.float32),
        scratch_types=[
            pltpu.VMEM((SEQ,), jnp.int32),
            pltpu.VMEM((SEQ, DIM), jnp.float32),
            pltpu.VMEM((SEQ, DIM), jnp.float32),
            pltpu.SemaphoreType.DMA,
        ],
    )
    def k(ids_hbm, table_hbm, pe_hbm, out_hbm, idx_v, rows_v, pe_v, sem):
        wid = lax.axis_index("s") * 2 + lax.axis_index("c")
        pltpu.sync_copy(pe_hbm, pe_v)

        def seq_body(t, carry):
            base = (wid * SEQ_PER_W + t) * SEQ
            pltpu.sync_copy(ids_hbm.at[pl.ds(base, SEQ)], idx_v)
            cp0 = pltpu.async_copy(
                table_hbm.at[idx_v.at[pl.ds(0, SPLIT)]],
                rows_v.at[pl.ds(0, SPLIT)], sem)
            cp1 = pltpu.async_copy(
                table_hbm.at[idx_v.at[pl.ds(SPLIT, SEQ - SPLIT)]],
                rows_v.at[pl.ds(SPLIT, SEQ - SPLIT)], sem)
            cp0.wait()
            cp1.wait()

            def row_body(r, rcarry):
                for j in range(DIM // LANES):
                    plsc.addupdate(rows_v.at[r, pl.ds(j * LANES, LANES)],
                                   pe_v[r, pl.ds(j * LANES, LANES)])
                return rcarry

            lax.fori_loop(0, SEQ, row_body, 0)
            pltpu.sync_copy(rows_v, out_hbm.at[pl.ds(base, SEQ)])
            return carry

        lax.fori_loop(0, SEQ_PER_W, seq_body, 0)

    return k


def kernel(input_ids, table, pos_enc):
    ids_flat = input_ids.reshape(-1).astype(jnp.int32)
    pe = pos_enc.reshape(SEQ, DIM)
    out = _make_kernel()(ids_flat, table, pe)
    return out.reshape(BATCH, SEQ, DIM)


# speedup vs baseline: 1.0436x; 1.0436x over previous
import functools

import jax
import jax.numpy as jnp
from jax import lax
from jax.experimental import pallas as pl
from jax.experimental.pallas import tpu as pltpu
from jax.experimental.pallas import tpu_sc as plsc

DIM = 64
SEQ = 200
BATCH = 1024
NW = 32
SEQ_PER_W = BATCH // NW
SPLIT = 128
LANES = 16
NBUF = 4


def _make_kernel():
    mesh = plsc.VectorSubcoreMesh(core_axis_name="c", subcore_axis_name="s")

    scratch = (
        [pltpu.VMEM((SEQ,), jnp.int32) for _ in range(NBUF)]
        + [pltpu.VMEM((SEQ, DIM), jnp.float32) for _ in range(NBUF)]
        + [pltpu.VMEM((SEQ, DIM), jnp.float32)]
        + [pltpu.SemaphoreType.DMA for _ in range(2 * NBUF)]
    )

    @functools.partial(
        pl.kernel,
        mesh=mesh,
        compiler_params=pltpu.CompilerParams(use_tc_tiling_on_sc=False),
        out_type=jax.ShapeDtypeStruct((BATCH * SEQ, DIM), jnp.float32),
        scratch_types=scratch,
    )
    def k(ids_hbm, table_hbm, pe_hbm, out_hbm, *refs):
        idxs = refs[0:NBUF]
        bufs = refs[NBUF:2 * NBUF]
        pe_v = refs[2 * NBUF]
        gsems = refs[2 * NBUF + 1: 2 * NBUF + 1 + NBUF]
        osems = refs[2 * NBUF + 1 + NBUF: 2 * NBUF + 1 + 2 * NBUF]

        wid = lax.axis_index("s") * 2 + lax.axis_index("c")
        base_row = wid * SEQ_PER_W * SEQ
        pltpu.sync_copy(pe_hbm, pe_v)

        def start_gather(t, b):
            base = base_row + t * SEQ
            pltpu.sync_copy(ids_hbm.at[pl.ds(base, SEQ)], idxs[b])
            pltpu.async_copy(table_hbm.at[idxs[b].at[pl.ds(0, SPLIT)]],
                             bufs[b].at[pl.ds(0, SPLIT)], gsems[b])
            pltpu.async_copy(table_hbm.at[idxs[b].at[pl.ds(SPLIT, SEQ - SPLIT)]],
                             bufs[b].at[pl.ds(SPLIT, SEQ - SPLIT)], gsems[b])

        def wait_gather(b):
            pltpu.make_async_copy(table_hbm.at[pl.ds(0, SEQ)], bufs[b],
                                  gsems[b]).wait()

        def add_pe(b):
            buf = bufs[b]

            @plsc.parallel_loop(0, SEQ, unroll=8)
            def _(r):
                for j in range(DIM // LANES):
                    plsc.addupdate(buf.at[r, pl.ds(j * LANES, LANES)],
                                   pe_v[r, pl.ds(j * LANES, LANES)])

        def process(t, b, regather):
            wait_gather(b)
            add_pe(b)
            cp = pltpu.async_copy(bufs[b], out_hbm.at[pl.ds(base_row + t * SEQ,
                                                            SEQ)], osems[b])
            if regather:
                cp.wait()
                start_gather(t + NBUF, b)
            return cp

        for b in range(NBUF):
            start_gather(b, b)

        @pl.loop(0, SEQ_PER_W - NBUF, step=NBUF)
        def _(g):
            for b in range(NBUF):
                process(g + b, b, regather=True)

        tail = []
        for b in range(NBUF):
            tail.append(process(SEQ_PER_W - NBUF + b, b, regather=False))
        for cp in tail:
            cp.wait()

    return k


def kernel(input_ids, table, pos_enc):
    ids_flat = input_ids.reshape(-1).astype(jnp.int32)
    pe = pos_enc.reshape(SEQ, DIM)
    out = _make_kernel()(ids_flat, table, pe)
    return out.reshape(BATCH, SEQ, DIM)
